# baseline (speedup 1.0000x reference)
import functools

import jax
import jax.numpy as jnp
from jax import lax
from jax.experimental import pallas as pl
from jax.experimental.pallas import tpu as pltpu
from jax.experimental.pallas import tpu_sc as plsc

BATCH = 4096
HIST = 200
DIM = 32
VOCAB = 1000000
NC, NS = 2, 16
NW = NC * NS
PACK = 128 // DIM
WRM_ROWS = VOCAB // PACK
NBLK = VOCAB // 128
TAIL = VOCAB - NBLK * 128
BLK_PER_W = NBLK // NW
BLK_REM = NBLK - BLK_PER_W * NW
BB = BATCH // 128

_MESH = plsc.VectorSubcoreMesh(core_axis_name="c", subcore_axis_name="s")


def _iota16():
    return lax.iota(jnp.int32, 16)


def _transpose_block(src, dst):
    for l0 in range(0, 128, 16):
        lvec = _iota16() + l0
        rquart = lax.shift_right_logical(lvec, 2)
        lmod = lax.bitwise_and(lvec, 3) * DIM
        for half in range(2):
            dvecs = [
                lax.bitwise_and(_iota16() + half * 16 + d0, DIM - 1)
                for d0 in range(16)
            ]
            vals = [plsc.load_gather(src, [dv, lvec]) for dv in dvecs]
            for dv, v in zip(dvecs, vals):
                plsc.store_scatter(dst, [rquart, lmod + dv], v)


@functools.partial(
    pl.kernel,
    mesh=_MESH,
    out_type=jax.ShapeDtypeStruct((WRM_ROWS, 128), jnp.float32),
    scratch_types=[
        pltpu.VMEM((DIM, 128), jnp.float32),
        pltpu.VMEM((DIM, 128), jnp.float32),
        pltpu.VMEM((DIM, 128), jnp.float32),
        pltpu.VMEM((DIM, 128), jnp.float32),
        pltpu.SemaphoreType.DMA,
        pltpu.SemaphoreType.DMA,
        pltpu.SemaphoreType.DMA,
        pltpu.SemaphoreType.DMA,
    ],
    compiler_params=pltpu.CompilerParams(needs_layout_passes=False),
)
def _transpose_table(w_t, wt_tail, wrm, s0, s1, d0, d1, si0, si1, so0, so1):
    wid = lax.axis_index("s") * NC + lax.axis_index("c")

    def in_slice(c):
        return w_t.at[:, pl.ds(pl.multiple_of(c * 128, 128), 128)]

    def out_slice(c):
        return wrm.at[pl.ds(pl.multiple_of(c * DIM, DIM), DIM), :]

    def blk(t):
        return t * NW + wid

    pltpu.async_copy(in_slice(blk(0)), s0, si0)
    pltpu.async_copy(in_slice(blk(1)), s1, si1)

    def pair_body(p, _):
        t0 = 2 * p

        def stage(t, s, d, si, so):
            pltpu.make_async_copy(in_slice(blk(t)), s, si).wait()

            @pl.when(p > 0)
            def _():
                pltpu.make_async_copy(d, out_slice(blk(t - 2)), so).wait()

            _transpose_block(s, d)
            pltpu.async_copy(d, out_slice(blk(t)), so)

            @pl.when(t + 2 < BLK_PER_W)
            def _():
                pltpu.async_copy(in_slice(blk(t + 2)), s, si)

        stage(t0, s0, d0, si0, so0)
        stage(t0 + 1, s1, d1, si1, so1)
        return 0

    lax.fori_loop(0, BLK_PER_W // 2, pair_body, 0)
    pltpu.make_async_copy(d0, out_slice(blk(BLK_PER_W - 2)), so0).wait()
    pltpu.make_async_copy(d1, out_slice(blk(BLK_PER_W - 1)), so1).wait()

    @pl.when(wid < BLK_REM)
    def _():
        c = BLK_PER_W * NW + wid
        pltpu.sync_copy(in_slice(c), s0)
        _transpose_block(s0, d0)
        pltpu.sync_copy(d0, out_slice(c))

    @pl.when(wid == NW - 1)
    def _():
        nrow = TAIL // PACK
        pltpu.sync_copy(wt_tail, s1.at[pl.ds(0, nrow), :])
        pltpu.sync_copy(
            s1.at[pl.ds(0, nrow), :], wrm.at[pl.ds(WRM_ROWS - nrow, nrow), :]
        )


@functools.partial(
    pl.kernel,
    mesh=_MESH,
    out_type=jax.ShapeDtypeStruct((HIST, PACK, BB, 8, 128), jnp.float32),
    scratch_types=[
        [pltpu.VMEM((128,), jnp.int32)] * 4,
        [pltpu.VMEM((128, DIM), jnp.float32)] * 4,
        [pltpu.VMEM((PACK, 8, 128), jnp.float32)] * 4,
        [pltpu.SemaphoreType.DMA] * 4,
        [pltpu.SemaphoreType.DMA] * 4,
        [pltpu.SemaphoreType.DMA] * 4,
    ],
    compiler_params=pltpu.CompilerParams(
        use_tc_tiling_on_sc=False, needs_layout_passes=False
    ),
)
def _gather(tok, wlin, out, iv, gv, ov, sI, sG, sO):
    wid = lax.axis_index("s") * NC + lax.axis_index("c")

    def tok_slice(h):
        return tok.at[pl.ds(pl.multiple_of(h * BATCH + wid * 128, 128), 128)]

    def out_slice(h):
        return out.at[h, :, wid]

    def extract(g, o):
        for l0 in range(0, 128, 16):
            lvec = _iota16() + l0
            for half in range(2):
                dvecs = [
                    lax.bitwise_and(_iota16() + half * 16 + d0, DIM - 1)
                    for d0 in range(16)
                ]
                vals = [plsc.load_gather(g, [lvec, dv]) for dv in dvecs]
                for dv, v in zip(dvecs, vals):
                    plsc.store_scatter(
                        o,
                        [
                            lax.shift_right_logical(dv, 3),
                            lax.bitwise_and(dv, 7),
                            lvec,
                        ],
                        v,
                    )

    for k in range(4):
        pltpu.async_copy(tok_slice(k), iv[k], sI[k])
    for k in range(2):
        pltpu.make_async_copy(tok_slice(k), iv[k], sI[k]).wait()
        pltpu.async_copy(wlin.at[iv[k]], gv[k], sG[k])

    def quad_body(q, _):
        h0 = 4 * q
        for k in range(4):
            h = h0 + k
            k2 = (k + 2) % 4
            @pl.when(h + 2 < HIST)
            def _():
                pltpu.make_async_copy(tok_slice(h + 2), iv[k2], sI[k2]).wait()
                pltpu.async_copy(wlin.at[iv[k2]], gv[k2], sG[k2])

            pltpu.make_async_copy(wlin.at[iv[k]], gv[k], sG[k]).wait()

            @pl.when(h + 4 < HIST)
            def _():
                pltpu.async_copy(tok_slice(h + 4), iv[k], sI[k])

            @pl.when(q > 0)
            def _():
                pltpu.make_async_copy(ov[k], out_slice(h - 4), sO[k]).wait()

            extract(gv[k], ov[k])
            pltpu.async_copy(ov[k], out_slice(h), sO[k])
        return 0

    lax.fori_loop(0, HIST // 4, quad_body, 0)
    for k in range(4):
        pltpu.make_async_copy(ov[k], out_slice(HIST - 4 + k), sO[k]).wait()


def kernel(token_ids, weight):
    wt_tail = weight[NBLK * 128 :].reshape(TAIL // PACK, 128)
    wrm = _transpose_table(weight.T, wt_tail)
    wlin = wrm.reshape(VOCAB, DIM)
    tok = token_ids.T.reshape(BATCH * HIST)
    out5 = _gather(tok, wlin)
    return out5.transpose(2, 4, 0, 1, 3).reshape(BATCH, HIST, DIM)


# speedup vs baseline: 1.7124x; 1.7124x over previous
import functools

import jax
import jax.numpy as jnp
from jax import lax
from jax.experimental import pallas as pl
from jax.experimental.pallas import tpu as pltpu
from jax.experimental.pallas import tpu_sc as plsc

BATCH = 4096
HIST = 200
DIM = 32
VOCAB = 1000000
NC, NS = 2, 16
NW = NC * NS
PACK = 128 // DIM
WRM_ROWS = VOCAB // PACK
NBLK = VOCAB // 128
TAIL = VOCAB - NBLK * 128
BLK_PER_W = NBLK // NW
BLK_REM = NBLK - BLK_PER_W * NW
BB = BATCH // 128

_MESH = plsc.VectorSubcoreMesh(core_axis_name="c", subcore_axis_name="s")


def _iota16():
    return lax.iota(jnp.int32, 16)


def _transpose_block(src, dst, sub):
    for l0 in range(0, 128, 16):
        lvec = _iota16() + l0
        gcol = lvec + sub * 128
        rquart = lax.shift_right_logical(lvec, 2) + sub * DIM
        lmod = lax.bitwise_and(lvec, 3) * DIM
        for half in range(2):
            dvecs = [
                lax.bitwise_and(_iota16() + half * 16 + d0, DIM - 1)
                for d0 in range(16)
            ]
            vals = [plsc.load_gather(src, [dv, gcol]) for dv in dvecs]
            for dv, v in zip(dvecs, vals):
                plsc.store_scatter(dst, [rquart, lmod + dv], v)


def _transpose_super(src, dst):
    def body(sub, carry):
        _transpose_block(src, dst, sub)
        return carry

    lax.fori_loop(0, PACK, body, 0)


@functools.partial(
    pl.kernel,
    mesh=_MESH,
    out_type=jax.ShapeDtypeStruct((WRM_ROWS, 128), jnp.float32),
    scratch_types=[
        pltpu.VMEM((DIM, 4 * 128), jnp.float32),
        pltpu.VMEM((DIM, 4 * 128), jnp.float32),
        pltpu.VMEM((4 * DIM, 128), jnp.float32),
        pltpu.VMEM((4 * DIM, 128), jnp.float32),
        pltpu.SemaphoreType.DMA,
        pltpu.SemaphoreType.DMA,
        pltpu.SemaphoreType.DMA,
        pltpu.SemaphoreType.DMA,
    ],
    compiler_params=pltpu.CompilerParams(needs_layout_passes=False),
)
def _transpose_table(w_t, wt_tail, wrm, s0, s1, d0, d1, si0, si1, so0, so1):
    wid = lax.axis_index("s") * NC + lax.axis_index("c")
    SUP_PER_W = 61

    def in_slice(C):
        return w_t.at[:, pl.ds(pl.multiple_of(C * 512, 512), 512)]

    def out_slice(C):
        return wrm.at[pl.ds(pl.multiple_of(C * 4 * DIM, 4 * DIM), 4 * DIM), :]

    def sup(t):
        return t * NW + wid

    pltpu.async_copy(in_slice(sup(0)), s0, si0)
    pltpu.async_copy(in_slice(sup(1)), s1, si1)

    def stage(t, first, s, d, si, so):
        pltpu.make_async_copy(in_slice(sup(t)), s, si).wait()

        @pl.when(jnp.logical_not(first))
        def _():
            pltpu.make_async_copy(d, out_slice(sup(t - 2)), so).wait()

        _transpose_super(s, d)
        pltpu.async_copy(d, out_slice(sup(t)), so)

        @pl.when(t + 2 < SUP_PER_W)
        def _():
            pltpu.async_copy(in_slice(sup(t + 2)), s, si)

    def pair_body(p, _):
        stage(2 * p, p == 0, s0, d0, si0, so0)
        stage(2 * p + 1, p == 0, s1, d1, si1, so1)
        return 0

    lax.fori_loop(0, SUP_PER_W // 2, pair_body, 0)
    stage(jnp.int32(SUP_PER_W - 1), jnp.bool_(False), s0, d0, si0, so0)
    pltpu.make_async_copy(d0, out_slice(sup(SUP_PER_W - 1)), so0).wait()
    pltpu.make_async_copy(d1, out_slice(sup(SUP_PER_W - 2)), so1).wait()

    @pl.when(wid < BLK_REM)
    def _():
        c = 1952 * 4 + wid
        pltpu.sync_copy(
            w_t.at[:, pl.ds(pl.multiple_of(c * 128, 128), 128)],
            s0.at[:, pl.ds(0, 128)],
        )
        _transpose_block(s0, d0, 0)
        pltpu.sync_copy(
            d0.at[pl.ds(0, DIM), :],
            wrm.at[pl.ds(pl.multiple_of(c * DIM, DIM), DIM), :],
        )

    @pl.when(wid == NW - 1)
    def _():
        nrow = TAIL // PACK
        pltpu.sync_copy(wt_tail, d1.at[pl.ds(0, nrow), :])
        pltpu.sync_copy(
            d1.at[pl.ds(0, nrow), :], wrm.at[pl.ds(WRM_ROWS - nrow, nrow), :]
        )


@functools.partial(
    pl.kernel,
    mesh=_MESH,
    out_type=jax.ShapeDtypeStruct((HIST, PACK, BB, 8, 128), jnp.float32),
    scratch_types=[
        [pltpu.VMEM((128,), jnp.int32)] * 4,
        [pltpu.VMEM((128, DIM), jnp.float32)] * 4,
        [pltpu.VMEM((PACK, 8, 128), jnp.float32)] * 4,
        [pltpu.SemaphoreType.DMA] * 4,
        [pltpu.SemaphoreType.DMA] * 4,
        [pltpu.SemaphoreType.DMA] * 4,
    ],
    compiler_params=pltpu.CompilerParams(
        use_tc_tiling_on_sc=False, needs_layout_passes=False
    ),
)
def _gather(tok, wlin, out, iv, gv, ov, sI, sG, sO):
    wid = lax.axis_index("s") * NC + lax.axis_index("c")

    def tok_slice(h):
        return tok.at[pl.ds(pl.multiple_of(h * BATCH + wid * 128, 128), 128)]

    def out_slice(h):
        return out.at[h, :, wid]

    def extract(g, o):
        for l0 in range(0, 128, 16):
            lvec = _iota16() + l0
            for half in range(2):
                dvecs = [
                    lax.bitwise_and(_iota16() + half * 16 + d0, DIM - 1)
                    for d0 in range(16)
                ]
                vals = [plsc.load_gather(g, [lvec, dv]) for dv in dvecs]
                for dv, v in zip(dvecs, vals):
                    plsc.store_scatter(
                        o,
                        [
                            lax.shift_right_logical(dv, 3),
                            lax.bitwise_and(dv, 7),
                            lvec,
                        ],
                        v,
                    )

    i0, i1 = iv[0], iv[1]
    g0, g1 = gv[0], gv[1]
    o0, o1 = ov[0], ov[1]
    sI0, sI1 = sI[0], sI[1]
    sG0, sG1 = sG[0], sG[1]
    sO0, sO1 = sO[0], sO[1]

    pltpu.async_copy(tok_slice(0), i0, sI0)
    pltpu.async_copy(tok_slice(1), i1, sI1)
    pltpu.make_async_copy(tok_slice(0), i0, sI0).wait()
    pltpu.async_copy(wlin.at[i0], g0, sG0)

    def stage(p, h, ivr, gb, ob, sIr, sGr, sOr):
        pltpu.make_async_copy(wlin.at[ivr], gb, sGr).wait()

        @pl.when(h + 2 < HIST)
        def _():
            pltpu.async_copy(tok_slice(h + 2), ivr, sIr)

        @pl.when(p > 0)
        def _():
            pltpu.make_async_copy(ob, out_slice(h - 2), sOr).wait()

        extract(gb, ob)
        pltpu.async_copy(ob, out_slice(h), sOr)

    def pair_body(p, _):
        h0 = 2 * p
        pltpu.make_async_copy(tok_slice(h0 + 1), i1, sI1).wait()
        pltpu.async_copy(wlin.at[i1], g1, sG1)
        stage(p, h0, i0, g0, o0, sI0, sG0, sO0)
        @pl.when(h0 + 2 < HIST)
        def _():
            pltpu.make_async_copy(tok_slice(h0 + 2), i0, sI0).wait()
            pltpu.async_copy(wlin.at[i0], g0, sG0)

        stage(p, h0 + 1, i1, g1, o1, sI1, sG1, sO1)
        return 0

    lax.fori_loop(0, HIST // 2, pair_body, 0)
    pltpu.make_async_copy(o0, out_slice(HIST - 2), sO0).wait()
    pltpu.make_async_copy(o1, out_slice(HIST - 1), sO1).wait()


def kernel(token_ids, weight):
    wt_tail = weight[NBLK * 128 :].reshape(TAIL // PACK, 128)
    wrm = _transpose_table(weight.T, wt_tail)
    wlin = wrm.reshape(VOCAB, DIM)
    tok = token_ids.T.reshape(BATCH * HIST)
    out5 = _gather(tok, wlin)
    return out5.transpose(2, 4, 0, 1, 3).reshape(BATCH, HIST, DIM)


# speedup vs baseline: 2.4647x; 1.4393x over previous
import functools

import jax
import jax.numpy as jnp
from jax import lax
from jax.experimental import pallas as pl
from jax.experimental.pallas import tpu as pltpu
from jax.experimental.pallas import tpu_sc as plsc

BATCH = 4096
HIST = 200
DIM = 32
VOCAB = 1000000
NC, NS = 2, 16
NW = NC * NS
PACK = 128 // DIM
WRM_ROWS = VOCAB // PACK
NBLK = VOCAB // 128
TAIL = VOCAB - NBLK * 128
BLK_PER_W = NBLK // NW
BLK_REM = NBLK - BLK_PER_W * NW
BB = BATCH // 128

_MESH = plsc.VectorSubcoreMesh(core_axis_name="c", subcore_axis_name="s")


def _iota16():
    return lax.iota(jnp.int32, 16)


def _transpose_block(src, dst, sub):
    for l0 in range(0, 128, 16):
        lvec = _iota16() + l0
        gcol = lvec + sub * 128
        rquart = lax.shift_right_logical(lvec, 2) + sub * DIM
        lmod = lax.bitwise_and(lvec, 3) * DIM
        for half in range(2):
            dvecs = [
                lax.bitwise_and(_iota16() + half * 16 + d0, DIM - 1)
                for d0 in range(16)
            ]
            vals = [plsc.load_gather(src, [dv, gcol]) for dv in dvecs]
            for dv, v in zip(dvecs, vals):
                plsc.store_scatter(dst, [rquart, lmod + dv], v)


def _transpose_super(src, dst):
    def body(sub, carry):
        _transpose_block(src, dst, sub)
        return carry

    lax.fori_loop(0, PACK, body, 0)


@functools.partial(
    pl.kernel,
    mesh=_MESH,
    out_type=jax.ShapeDtypeStruct((WRM_ROWS, 128), jnp.float32),
    scratch_types=[
        pltpu.VMEM((DIM, 4 * 128), jnp.float32),
        pltpu.VMEM((DIM, 4 * 128), jnp.float32),
        pltpu.VMEM((4 * DIM, 128), jnp.float32),
        pltpu.VMEM((4 * DIM, 128), jnp.float32),
        pltpu.SemaphoreType.DMA,
        pltpu.SemaphoreType.DMA,
        pltpu.SemaphoreType.DMA,
        pltpu.SemaphoreType.DMA,
    ],
    compiler_params=pltpu.CompilerParams(needs_layout_passes=False),
)
def _transpose_table(w_t, wt_tail, wrm, s0, s1, d0, d1, si0, si1, so0, so1):
    wid = lax.axis_index("s") * NC + lax.axis_index("c")
    SUP_PER_W = 61

    def in_slice(C):
        return w_t.at[:, pl.ds(pl.multiple_of(C * 512, 512), 512)]

    def out_slice(C):
        return wrm.at[pl.ds(pl.multiple_of(C * 4 * DIM, 4 * DIM), 4 * DIM), :]

    def sup(t):
        return t * NW + wid

    pltpu.async_copy(in_slice(sup(0)), s0, si0)
    pltpu.async_copy(in_slice(sup(1)), s1, si1)

    def stage(t, first, s, d, si, so):
        pltpu.make_async_copy(in_slice(sup(t)), s, si).wait()

        @pl.when(jnp.logical_not(first))
        def _():
            pltpu.make_async_copy(d, out_slice(sup(t - 2)), so).wait()

        _transpose_super(s, d)
        pltpu.async_copy(d, out_slice(sup(t)), so)

        @pl.when(t + 2 < SUP_PER_W)
        def _():
            pltpu.async_copy(in_slice(sup(t + 2)), s, si)

    def pair_body(p, _):
        stage(2 * p, p == 0, s0, d0, si0, so0)
        stage(2 * p + 1, p == 0, s1, d1, si1, so1)
        return 0

    lax.fori_loop(0, SUP_PER_W // 2, pair_body, 0)
    stage(jnp.int32(SUP_PER_W - 1), jnp.bool_(False), s0, d0, si0, so0)
    pltpu.make_async_copy(d0, out_slice(sup(SUP_PER_W - 1)), so0).wait()
    pltpu.make_async_copy(d1, out_slice(sup(SUP_PER_W - 2)), so1).wait()

    @pl.when(wid < BLK_REM)
    def _():
        c = 1952 * 4 + wid
        pltpu.sync_copy(
            w_t.at[:, pl.ds(pl.multiple_of(c * 128, 128), 128)],
            s0.at[:, pl.ds(0, 128)],
        )
        _transpose_block(s0, d0, 0)
        pltpu.sync_copy(
            d0.at[pl.ds(0, DIM), :],
            wrm.at[pl.ds(pl.multiple_of(c * DIM, DIM), DIM), :],
        )

    @pl.when(wid == NW - 1)
    def _():
        nrow = TAIL // PACK
        pltpu.sync_copy(wt_tail, d1.at[pl.ds(0, nrow), :])
        pltpu.sync_copy(
            d1.at[pl.ds(0, nrow), :], wrm.at[pl.ds(WRM_ROWS - nrow, nrow), :]
        )


@functools.partial(
    pl.kernel,
    mesh=_MESH,
    out_type=jax.ShapeDtypeStruct((HIST, PACK, BB, 8, 128), jnp.float32),
    scratch_types=[
        [pltpu.VMEM((2, 128), jnp.int32)] * 2,
        [pltpu.VMEM((2 * 128, DIM), jnp.float32)] * 2,
        [pltpu.VMEM((2 * PACK, 8, 128), jnp.float32)] * 2,
        [pltpu.SemaphoreType.DMA] * 2,
        [pltpu.SemaphoreType.DMA] * 2,
        [pltpu.SemaphoreType.DMA] * 2,
    ],
    compiler_params=pltpu.CompilerParams(
        use_tc_tiling_on_sc=False, needs_layout_passes=False
    ),
)
def _gather(tok3, wlin, out, iv, gv, ov, sI, sG, sO):
    wid = lax.axis_index("s") * NC + lax.axis_index("c")
    M = HIST // 2

    def tok_slice(m):
        return tok3.at[pl.ds(pl.multiple_of(2 * m, 2), 2), wid]

    def out_slice(m, j):
        return out.at[2 * m + j, :, wid]

    def fire_gathers(k, sGk):
        pltpu.async_copy(wlin.at[iv[k].at[0]], gv[k].at[pl.ds(0, 128)], sGk)
        pltpu.async_copy(wlin.at[iv[k].at[1]], gv[k].at[pl.ds(128, 128)], sGk)

    def wait_gathers(k, sGk):
        pltpu.make_async_copy(
            wlin.at[iv[k].at[0]], gv[k].at[pl.ds(0, 128)], sGk
        ).wait()
        pltpu.make_async_copy(
            wlin.at[iv[k].at[1]], gv[k].at[pl.ds(128, 128)], sGk
        ).wait()

    def extract(g, o, j):
        def l_body(li, carry):
            lvec = _iota16() + li * 16
            grow = lvec + j * 128
            for half in range(2):
                dvecs = [
                    lax.bitwise_and(_iota16() + half * 16 + d0, DIM - 1)
                    for d0 in range(16)
                ]
                vals = [plsc.load_gather(g, [grow, dv]) for dv in dvecs]
                for dv, v in zip(dvecs, vals):
                    plsc.store_scatter(
                        o,
                        [
                            lax.shift_right_logical(dv, 3) + 4 * j,
                            lax.bitwise_and(dv, 7),
                            lvec,
                        ],
                        v,
                    )
            return carry

        lax.fori_loop(0, 8, l_body, 0)

    pltpu.async_copy(tok_slice(0), iv[0], sI[0])
    pltpu.async_copy(tok_slice(1), iv[1], sI[1])
    pltpu.make_async_copy(tok_slice(0), iv[0], sI[0]).wait()
    fire_gathers(0, sG[0])

    def stage(p, m, k):
        wait_gathers(k, sG[k])

        @pl.when(m + 2 < M)
        def _():
            pltpu.async_copy(tok_slice(m + 2), iv[k], sI[k])

        @pl.when(p > 0)
        def _():
            pltpu.make_async_copy(
                ov[k].at[pl.ds(0, PACK)], out_slice(m - 2, 0), sO[k]
            ).wait()
            pltpu.make_async_copy(
                ov[k].at[pl.ds(PACK, PACK)], out_slice(m - 2, 1), sO[k]
            ).wait()

        extract(gv[k], ov[k], 0)
        extract(gv[k], ov[k], 1)
        pltpu.async_copy(ov[k].at[pl.ds(0, PACK)], out_slice(m, 0), sO[k])
        pltpu.async_copy(ov[k].at[pl.ds(PACK, PACK)], out_slice(m, 1), sO[k])

    def pair_body(p, _):
        m0 = 2 * p
        pltpu.make_async_copy(tok_slice(m0 + 1), iv[1], sI[1]).wait()
        fire_gathers(1, sG[1])
        stage(p, m0, 0)
        @pl.when(m0 + 2 < M)
        def _():
            pltpu.make_async_copy(tok_slice(m0 + 2), iv[0], sI[0]).wait()
            fire_gathers(0, sG[0])

        stage(p, m0 + 1, 1)
        return 0

    lax.fori_loop(0, M // 2, pair_body, 0)
    for k, m in ((0, M - 2), (1, M - 1)):
        pltpu.make_async_copy(
            ov[k].at[pl.ds(0, PACK)], out_slice(m, 0), sO[k]
        ).wait()
        pltpu.make_async_copy(
            ov[k].at[pl.ds(PACK, PACK)], out_slice(m, 1), sO[k]
        ).wait()


def kernel(token_ids, weight):
    wt_tail = weight[NBLK * 128 :].reshape(TAIL // PACK, 128)
    wrm = _transpose_table(weight.T, wt_tail)
    wlin = wrm.reshape(VOCAB, DIM)
    tok3 = token_ids.T.reshape(HIST, BB, 128)
    out5 = _gather(tok3, wlin)
    return out5.transpose(2, 4, 0, 1, 3).reshape(BATCH, HIST, DIM)


# speedup vs baseline: 2.4860x; 1.0086x over previous
import functools

import jax
import jax.numpy as jnp
from jax import lax
from jax.experimental import pallas as pl
from jax.experimental.pallas import tpu as pltpu
from jax.experimental.pallas import tpu_sc as plsc

BATCH = 4096
HIST = 200
DIM = 32
VOCAB = 1000000
NC, NS = 2, 16
NW = NC * NS
PACK = 128 // DIM
WRM_ROWS = VOCAB // PACK
NBLK = VOCAB // 128
TAIL = VOCAB - NBLK * 128
BLK_PER_W = NBLK // NW
BLK_REM = NBLK - BLK_PER_W * NW
BB = BATCH // 128

_MESH = plsc.VectorSubcoreMesh(core_axis_name="c", subcore_axis_name="s")


def _iota16():
    return lax.iota(jnp.int32, 16)


def _transpose_block(src, dst, sub):
    for l0 in range(0, 128, 16):
        lvec = _iota16() + l0
        gcol = lvec + sub * 128
        rquart = lax.shift_right_logical(lvec, 2) + sub * DIM
        lmod = lax.bitwise_and(lvec, 3) * DIM
        for half in range(2):
            dvecs = [
                lax.bitwise_and(_iota16() + half * 16 + d0, DIM - 1)
                for d0 in range(16)
            ]
            vals = [plsc.load_gather(src, [dv, gcol]) for dv in dvecs]
            for dv, v in zip(dvecs, vals):
                plsc.store_scatter(dst, [rquart, lmod + dv], v)


def _transpose_super(src, dst):
    def body(sub, carry):
        _transpose_block(src, dst, sub)
        return carry

    lax.fori_loop(0, PACK, body, 0)


@functools.partial(
    pl.kernel,
    mesh=_MESH,
    out_type=(
        jax.ShapeDtypeStruct((WRM_ROWS, 128), jnp.float32),
        jax.ShapeDtypeStruct((BATCH * HIST,), jnp.int32),
    ),
    scratch_types=[
        pltpu.VMEM((8, BATCH), jnp.int32),
        pltpu.VMEM((DIM, 4 * 128), jnp.float32),
        pltpu.VMEM((DIM, 4 * 128), jnp.float32),
        pltpu.VMEM((4 * DIM, 128), jnp.float32),
        pltpu.VMEM((4 * DIM, 128), jnp.float32),
        pltpu.SemaphoreType.DMA,
        pltpu.SemaphoreType.DMA,
        pltpu.SemaphoreType.DMA,
        pltpu.SemaphoreType.DMA,
    ],
    compiler_params=pltpu.CompilerParams(needs_layout_passes=False),
)
def _transpose_table(
    w_t, wt_tail, tok_t, wrm, tok_lin, tbuf, s0, s1, d0, d1, si0, si1, so0, so1
):
    wid = lax.axis_index("s") * NC + lax.axis_index("c")
    SUP_PER_W = 61

    def in_slice(C):
        return w_t.at[:, pl.ds(pl.multiple_of(C * 512, 512), 512)]

    def out_slice(C):
        return wrm.at[pl.ds(pl.multiple_of(C * 4 * DIM, 4 * DIM), 4 * DIM), :]

    def sup(t):
        return t * NW + wid

    pltpu.async_copy(in_slice(sup(0)), s0, si0)
    pltpu.async_copy(in_slice(sup(1)), s1, si1)

    def stage(t, first, s, d, si, so):
        pltpu.make_async_copy(in_slice(sup(t)), s, si).wait()

        @pl.when(jnp.logical_not(first))
        def _():
            pltpu.make_async_copy(d, out_slice(sup(t - 2)), so).wait()

        _transpose_super(s, d)
        pltpu.async_copy(d, out_slice(sup(t)), so)

        @pl.when(t + 2 < SUP_PER_W)
        def _():
            pltpu.async_copy(in_slice(sup(t + 2)), s, si)

    def pair_body(p, _):
        stage(2 * p, p == 0, s0, d0, si0, so0)
        stage(2 * p + 1, p == 0, s1, d1, si1, so1)
        return 0

    lax.fori_loop(0, SUP_PER_W // 2, pair_body, 0)
    stage(jnp.int32(SUP_PER_W - 1), jnp.bool_(False), s0, d0, si0, so0)
    pltpu.make_async_copy(d0, out_slice(sup(SUP_PER_W - 1)), so0).wait()
    pltpu.make_async_copy(d1, out_slice(sup(SUP_PER_W - 2)), so1).wait()

    @pl.when(wid < BLK_REM)
    def _():
        c = 1952 * 4 + wid
        pltpu.sync_copy(
            w_t.at[:, pl.ds(pl.multiple_of(c * 128, 128), 128)],
            s0.at[:, pl.ds(0, 128)],
        )
        _transpose_block(s0, d0, 0)
        pltpu.sync_copy(
            d0.at[pl.ds(0, DIM), :],
            wrm.at[pl.ds(pl.multiple_of(c * DIM, DIM), DIM), :],
        )

    @pl.when(wid == NW - 1)
    def _():
        nrow = TAIL // PACK
        pltpu.sync_copy(wt_tail, d1.at[pl.ds(0, nrow), :])
        pltpu.sync_copy(
            d1.at[pl.ds(0, nrow), :], wrm.at[pl.ds(WRM_ROWS - nrow, nrow), :]
        )

    @pl.when(wid < HIST // 8)
    def _():
        pltpu.sync_copy(
            tok_t.at[pl.ds(pl.multiple_of(wid * 8, 8), 8), :], tbuf
        )
        for sr in range(8):
            pltpu.sync_copy(
                tbuf.at[sr],
                tok_lin.at[
                    pl.ds(pl.multiple_of((wid * 8 + sr) * BATCH, 8), BATCH)
                ],
            )


@functools.partial(
    pl.kernel,
    mesh=_MESH,
    out_type=jax.ShapeDtypeStruct((HIST, PACK, BB, 8, 128), jnp.float32),
    scratch_types=[
        [pltpu.VMEM((2, 128), jnp.int32)] * 2,
        [pltpu.VMEM((2 * 128, DIM), jnp.float32)] * 2,
        [pltpu.VMEM((2 * PACK, 8, 128), jnp.float32)] * 2,
        [pltpu.SemaphoreType.DMA] * 2,
        [pltpu.SemaphoreType.DMA] * 2,
        [pltpu.SemaphoreType.DMA] * 2,
    ],
    compiler_params=pltpu.CompilerParams(
        use_tc_tiling_on_sc=False, needs_layout_passes=False
    ),
)
def _gather(tok3, wlin, out, iv, gv, ov, sI, sG, sO):
    wid = lax.axis_index("s") * NC + lax.axis_index("c")
    M = HIST // 2

    def tok_slice(m):
        return tok3.at[pl.ds(pl.multiple_of(2 * m, 2), 2), wid]

    def out_slice(m, j):
        return out.at[2 * m + j, :, wid]

    def fire_gathers(k, sGk):
        pltpu.async_copy(wlin.at[iv[k].at[0]], gv[k].at[pl.ds(0, 128)], sGk)
        pltpu.async_copy(wlin.at[iv[k].at[1]], gv[k].at[pl.ds(128, 128)], sGk)

    def wait_gathers(k, sGk):
        pltpu.make_async_copy(
            wlin.at[iv[k].at[0]], gv[k].at[pl.ds(0, 128)], sGk
        ).wait()
        pltpu.make_async_copy(
            wlin.at[iv[k].at[1]], gv[k].at[pl.ds(128, 128)], sGk
        ).wait()

    def extract(g, o, j):
        def l_body(li, carry):
            lvec = _iota16() + li * 16
            grow = lvec + j * 128
            for half in range(2):
                dvecs = [
                    lax.bitwise_and(_iota16() + half * 16 + d0, DIM - 1)
                    for d0 in range(16)
                ]
                vals = [plsc.load_gather(g, [grow, dv]) for dv in dvecs]
                for dv, v in zip(dvecs, vals):
                    plsc.store_scatter(
                        o,
                        [
                            lax.shift_right_logical(dv, 3) + 4 * j,
                            lax.bitwise_and(dv, 7),
                            lvec,
                        ],
                        v,
                    )
            return carry

        lax.fori_loop(0, 8, l_body, 0)

    pltpu.async_copy(tok_slice(0), iv[0], sI[0])
    pltpu.async_copy(tok_slice(1), iv[1], sI[1])
    pltpu.make_async_copy(tok_slice(0), iv[0], sI[0]).wait()
    fire_gathers(0, sG[0])

    def stage(p, m, k):
        wait_gathers(k, sG[k])

        @pl.when(m + 2 < M)
        def _():
            pltpu.async_copy(tok_slice(m + 2), iv[k], sI[k])

        @pl.when(p > 0)
        def _():
            pltpu.make_async_copy(
                ov[k].at[pl.ds(0, PACK)], out_slice(m - 2, 0), sO[k]
            ).wait()
            pltpu.make_async_copy(
                ov[k].at[pl.ds(PACK, PACK)], out_slice(m - 2, 1), sO[k]
            ).wait()

        extract(gv[k], ov[k], 0)
        extract(gv[k], ov[k], 1)
        pltpu.async_copy(ov[k].at[pl.ds(0, PACK)], out_slice(m, 0), sO[k])
        pltpu.async_copy(ov[k].at[pl.ds(PACK, PACK)], out_slice(m, 1), sO[k])

    def pair_body(p, _):
        m0 = 2 * p
        pltpu.make_async_copy(tok_slice(m0 + 1), iv[1], sI[1]).wait()
        fire_gathers(1, sG[1])
        stage(p, m0, 0)
        @pl.when(m0 + 2 < M)
        def _():
            pltpu.make_async_copy(tok_slice(m0 + 2), iv[0], sI[0]).wait()
            fire_gathers(0, sG[0])

        stage(p, m0 + 1, 1)
        return 0

    lax.fori_loop(0, M // 2, pair_body, 0)
    for k, m in ((0, M - 2), (1, M - 1)):
        pltpu.make_async_copy(
            ov[k].at[pl.ds(0, PACK)], out_slice(m, 0), sO[k]
        ).wait()
        pltpu.make_async_copy(
            ov[k].at[pl.ds(PACK, PACK)], out_slice(m, 1), sO[k]
        ).wait()


def kernel(token_ids, weight):
    wt_tail = weight[NBLK * 128 :].reshape(TAIL // PACK, 128)
    wrm, tok_lin = _transpose_table(weight.T, wt_tail, token_ids.T)
    wlin = wrm.reshape(VOCAB, DIM)
    tok3 = tok_lin.reshape(HIST, BB, 128)
    out5 = _gather(tok3, wlin)
    return out5.transpose(2, 4, 0, 1, 3).reshape(BATCH, HIST, DIM)
